# baseline (speedup 1.0000x reference)
import functools

import jax
import jax.numpy as jnp
from jax import lax
from jax.experimental import pallas as pl
from jax.experimental.pallas import tpu as pltpu
from jax.experimental.pallas import tpu_sc as plsc

VOCAB = 100000
DIM = 64
BATCH = 16384

NC = 2
NS = 16
L = 16
NW = NC * NS
DPW = DIM // NW
E = 2048
NE = BATCH // E
UN = 1
EG = E // L // UN

_mesh = plsc.VectorSubcoreMesh(core_axis_name="c", subcore_axis_name="s")


@functools.partial(
    pl.kernel,
    out_type=jax.ShapeDtypeStruct((NC, BATCH), jnp.float32),
    mesh=_mesh,
    scratch_types=[
        pltpu.VMEM((1, VOCAB), jnp.float32),
        pltpu.VMEM((BATCH,), jnp.float32),
        pltpu.VMEM((E,), jnp.int32),
        pltpu.VMEM((E,), jnp.int32),
        pltpu.VMEM((E,), jnp.float32),
        pltpu.VMEM((E,), jnp.int32),
        pltpu.VMEM_SHARED((BATCH,), jnp.float32),
        pltpu.SemaphoreType.DMA,
        pltpu.SemaphoreType.DMA,
        pltpu.SemaphoreType.DMA,
    ],
    compiler_params=pltpu.CompilerParams(
        needs_layout_passes=False,
        use_tc_tiling_on_sc=True,
    ),
)
def _sc_dot(item_ids_hbm, ctx_ids_hbm, itemT_hbm, ctxT_hbm, out_hbm,
            row_v, vals_v, ids0_v, ids1_v, prod_v, idx_v, acc_sh,
            sem, isem0, isem1):
    cid = lax.axis_index("c")
    sid = lax.axis_index("s")
    wid = sid * NC + cid

    lanes = lax.broadcasted_iota(jnp.int32, (L,), 0)
    zrow = jnp.zeros((L,), jnp.int32)

    def idx_g(g, carry):
        base = pl.multiple_of(g * L * UN, L)
        for u in range(UN):
            o = base + u * L
            idx_v[pl.ds(o, L)] = lanes + o
        return carry
    lax.fori_loop(0, EG, idx_g, 0)

    @pl.when(sid == 0)
    def _():
        def zero_g(g, carry):
            base = pl.multiple_of(g * L * UN, L)
            for u in range(UN):
                prod_v[pl.ds(base + u * L, L)] = jnp.zeros((L,), jnp.float32)
            return carry
        lax.fori_loop(0, EG, zero_g, 0)
        for ck in range(NE):
            pltpu.sync_copy(prod_v, acc_sh.at[pl.ds(ck * E, E)])

    plsc.subcore_barrier()

    for di in range(DPW):
        d = wid * DPW + di

        ibufs = (ids0_v, ids1_v)
        isems = (isem0, isem1)

        rcp = pltpu.async_copy(itemT_hbm.at[pl.ds(d, 1), :], row_v, sem)
        ih = {0: pltpu.async_copy(item_ids_hbm.at[pl.ds(0, E)], ibufs[0],
                                  isems[0])}
        rcp.wait()
        for ck in range(NE):
            if ck + 1 < NE:
                ih[ck + 1] = pltpu.async_copy(
                    item_ids_hbm.at[pl.ds((ck + 1) * E, E)],
                    ibufs[(ck + 1) % 2], isems[(ck + 1) % 2])
            ih.pop(ck).wait()
            ids_v = ibufs[ck % 2]

            @plsc.parallel_loop(0, EG, 1, unroll=16)
            def item_g(g):
                base = pl.multiple_of(g * L * UN, L)
                for u in range(UN):
                    o = base + u * L
                    v = ids_v[pl.ds(o, L)]
                    x = plsc.load_gather(row_v, [zrow, v])
                    vals_v[pl.ds(ck * E + o, L)] = x

        rcp = pltpu.async_copy(ctxT_hbm.at[pl.ds(d, 1), :], row_v, sem)
        ih = {0: pltpu.async_copy(ctx_ids_hbm.at[pl.ds(0, E)], ibufs[0],
                                  isems[0])}
        rcp.wait()
        for ck in range(NE):
            if ck + 1 < NE:
                ih[ck + 1] = pltpu.async_copy(
                    ctx_ids_hbm.at[pl.ds((ck + 1) * E, E)],
                    ibufs[(ck + 1) % 2], isems[(ck + 1) % 2])
            ih.pop(ck).wait()
            ids_v = ibufs[ck % 2]

            @plsc.parallel_loop(0, EG, 1, unroll=16)
            def ctx_g(g):
                base = pl.multiple_of(g * L * UN, L)
                for u in range(UN):
                    o = base + u * L
                    v = ids_v[pl.ds(o, L)]
                    y = plsc.load_gather(row_v, [zrow, v])
                    x = vals_v[pl.ds(ck * E + o, L)]
                    prod_v[pl.ds(o, L)] = x * y
            pltpu.sync_copy(prod_v,
                            acc_sh.at[pl.ds(ck * E, E)].at[idx_v], add=True)

    plsc.subcore_barrier()

    @pl.when(sid == 0)
    def _():
        for ck in range(NE):
            pltpu.sync_copy(acc_sh.at[pl.ds(ck * E, E)],
                            out_hbm.at[cid, pl.ds(ck * E, E)])


def kernel(item_ids, context_ids, item_table, context_table):
    partial = _sc_dot(
        item_ids.astype(jnp.int32),
        context_ids.astype(jnp.int32),
        item_table.T,
        context_table.T,
    )
    return partial[0] + partial[1]


# speedup vs baseline: 1.0074x; 1.0074x over previous
import functools

import jax
import jax.numpy as jnp
from jax import lax
from jax.experimental import pallas as pl
from jax.experimental.pallas import tpu as pltpu
from jax.experimental.pallas import tpu_sc as plsc

VOCAB = 100000
DIM = 64
BATCH = 16384

NC = 2
NS = 16
L = 16
NW = NC * NS
DPW = DIM // NW
E = 2048
NE = BATCH // E
UN = 2
EG = E // L // UN

_mesh = plsc.VectorSubcoreMesh(core_axis_name="c", subcore_axis_name="s")


@functools.partial(
    pl.kernel,
    out_type=jax.ShapeDtypeStruct((NC, BATCH), jnp.float32),
    mesh=_mesh,
    scratch_types=[
        pltpu.VMEM((1, VOCAB), jnp.float32),
        pltpu.VMEM((BATCH,), jnp.float32),
        pltpu.VMEM((E,), jnp.int32),
        pltpu.VMEM((E,), jnp.int32),
        pltpu.VMEM((E,), jnp.float32),
        pltpu.VMEM((E,), jnp.int32),
        pltpu.VMEM_SHARED((BATCH,), jnp.float32),
        pltpu.SemaphoreType.DMA,
        pltpu.SemaphoreType.DMA,
        pltpu.SemaphoreType.DMA,
    ],
    compiler_params=pltpu.CompilerParams(
        needs_layout_passes=False,
        use_tc_tiling_on_sc=True,
    ),
)
def _sc_dot(item_ids_hbm, ctx_ids_hbm, itemT_hbm, ctxT_hbm, out_hbm,
            row_v, vals_v, ids0_v, ids1_v, prod_v, idx_v, acc_sh,
            sem, isem0, isem1):
    cid = lax.axis_index("c")
    sid = lax.axis_index("s")
    wid = sid * NC + cid

    lanes = lax.broadcasted_iota(jnp.int32, (L,), 0)
    zrow = jnp.zeros((L,), jnp.int32)

    def idx_g(g, carry):
        base = pl.multiple_of(g * L * UN, L)
        for u in range(UN):
            o = base + u * L
            idx_v[pl.ds(o, L)] = lanes + o
        return carry
    lax.fori_loop(0, EG, idx_g, 0)

    @pl.when(sid == 0)
    def _():
        def zero_g(g, carry):
            base = pl.multiple_of(g * L * UN, L)
            for u in range(UN):
                prod_v[pl.ds(base + u * L, L)] = jnp.zeros((L,), jnp.float32)
            return carry
        lax.fori_loop(0, EG, zero_g, 0)
        for ck in range(NE):
            pltpu.sync_copy(prod_v, acc_sh.at[pl.ds(ck * E, E)])

    plsc.subcore_barrier()

    for di in range(DPW):
        d = wid * DPW + di

        ibufs = (ids0_v, ids1_v)
        isems = (isem0, isem1)

        rcp = pltpu.async_copy(itemT_hbm.at[pl.ds(d, 1), :], row_v, sem)
        ih = {0: pltpu.async_copy(item_ids_hbm.at[pl.ds(0, E)], ibufs[0],
                                  isems[0])}
        rcp.wait()
        for ck in range(NE):
            if ck + 1 < NE:
                ih[ck + 1] = pltpu.async_copy(
                    item_ids_hbm.at[pl.ds((ck + 1) * E, E)],
                    ibufs[(ck + 1) % 2], isems[(ck + 1) % 2])
            ih.pop(ck).wait()
            ids_v = ibufs[ck % 2]

            @plsc.parallel_loop(0, EG, 1, unroll=8)
            def item_g(g):
                base = pl.multiple_of(g * L * UN, L)
                for u in range(UN):
                    o = base + u * L
                    v = ids_v[pl.ds(o, L)]
                    x = plsc.load_gather(row_v, [zrow, v])
                    vals_v[pl.ds(ck * E + o, L)] = x

        rcp = pltpu.async_copy(ctxT_hbm.at[pl.ds(d, 1), :], row_v, sem)
        ih = {0: pltpu.async_copy(ctx_ids_hbm.at[pl.ds(0, E)], ibufs[0],
                                  isems[0])}
        rcp.wait()
        for ck in range(NE):
            if ck + 1 < NE:
                ih[ck + 1] = pltpu.async_copy(
                    ctx_ids_hbm.at[pl.ds((ck + 1) * E, E)],
                    ibufs[(ck + 1) % 2], isems[(ck + 1) % 2])
            ih.pop(ck).wait()
            ids_v = ibufs[ck % 2]

            @plsc.parallel_loop(0, EG, 1, unroll=8)
            def ctx_g(g):
                base = pl.multiple_of(g * L * UN, L)
                for u in range(UN):
                    o = base + u * L
                    v = ids_v[pl.ds(o, L)]
                    y = plsc.load_gather(row_v, [zrow, v])
                    x = vals_v[pl.ds(ck * E + o, L)]
                    prod_v[pl.ds(o, L)] = x * y
            pltpu.sync_copy(prod_v,
                            acc_sh.at[pl.ds(ck * E, E)].at[idx_v], add=True)

    plsc.subcore_barrier()

    @pl.when(sid == 0)
    def _():
        for ck in range(NE):
            pltpu.sync_copy(acc_sh.at[pl.ds(ck * E, E)],
                            out_hbm.at[cid, pl.ds(ck * E, E)])


def kernel(item_ids, context_ids, item_table, context_table):
    partial = _sc_dot(
        item_ids.astype(jnp.int32),
        context_ids.astype(jnp.int32),
        item_table.T,
        context_table.T,
    )
    return partial[0] + partial[1]
